# baseline (speedup 1.0000x reference)
import jax
import jax.numpy as jnp
from jax import lax
from jax.experimental import pallas as pl
from jax.experimental.pallas import tpu as pltpu
from jax.experimental.pallas import tpu_sc as plsc

VOCAB = 100000
EMBED_DIM = 32
BATCH = 1024

NC, NS = 2, 16
NW = NC * NS
B_PER_W = BATCH // NW


def _gather_body(table_hbm, idx_hbm, out_hbm, idx_v, rows_v, sem):
    wid = lax.axis_index("s") * NC + lax.axis_index("c")
    base = wid * B_PER_W
    pltpu.sync_copy(idx_hbm.at[pl.ds(base, B_PER_W)], idx_v)
    pltpu.async_copy(table_hbm.at[idx_v], rows_v, sem).wait()
    pltpu.sync_copy(rows_v, out_hbm.at[pl.ds(base, B_PER_W)])


_sc_gather = pl.kernel(
    _gather_body,
    mesh=plsc.VectorSubcoreMesh(core_axis_name="c", subcore_axis_name="s"),
    out_type=jax.ShapeDtypeStruct((BATCH, EMBED_DIM), jnp.float32),
    scratch_types=[
        pltpu.VMEM((B_PER_W,), jnp.int32),
        pltpu.VMEM((B_PER_W, EMBED_DIM), jnp.float32),
        pltpu.SemaphoreType.DMA,
    ],
    compiler_params=pltpu.CompilerParams(use_tc_tiling_on_sc=False),
)


TN = 4096
KA = EMBED_DIM + 1


def _proj_body(w_ref, emb_ref, out_ref):
    out_ref[...] = lax.dot_general(
        w_ref[...], emb_ref[...],
        dimension_numbers=(((0,), (1,)), ((), ())),
        preferred_element_type=jnp.float32,
    )


@jax.jit
def _run(idx, emb_table, W, b):
    embeds = _sc_gather(emb_table, idx)
    w_aug = jnp.concatenate([W.T, b[None, :]], axis=0)
    emb_aug = jnp.concatenate(
        [embeds, jnp.ones((BATCH, 1), jnp.float32)], axis=1)
    out_t = pl.pallas_call(
        _proj_body,
        grid=(pl.cdiv(VOCAB, TN),),
        in_specs=[
            pl.BlockSpec((KA, TN), lambda i: (0, i)),
            pl.BlockSpec((BATCH, KA), lambda i: (0, 0)),
        ],
        out_specs=pl.BlockSpec((TN, BATCH), lambda i: (i, 0)),
        out_shape=jax.ShapeDtypeStruct((VOCAB, BATCH), jnp.float32),
        compiler_params=pltpu.CompilerParams(
            dimension_semantics=("parallel",),
        ),
    )(w_aug, emb_aug)
    return out_t.T


def kernel(inputs, emb_table, W, b):
    return _run(inputs.astype(jnp.int32), emb_table, W, b)


# speedup vs baseline: 1.0485x; 1.0485x over previous
import jax
import jax.numpy as jnp
from jax import lax
from jax.experimental import pallas as pl
from jax.experimental.pallas import tpu as pltpu
from jax.experimental.pallas import tpu_sc as plsc

VOCAB = 100000
EMBED_DIM = 32
BATCH = 1024

NC, NS = 2, 16
NW = NC * NS
B_PER_W = BATCH // NW


def _gather_body(table_hbm, idx_hbm, out_hbm, idx_v, rows_v, sem):
    wid = lax.axis_index("s") * NC + lax.axis_index("c")
    base = wid * B_PER_W
    pltpu.sync_copy(idx_hbm.at[pl.ds(base, B_PER_W)], idx_v)
    pltpu.async_copy(table_hbm.at[idx_v], rows_v, sem).wait()
    pltpu.sync_copy(rows_v, out_hbm.at[pl.ds(base, B_PER_W)])


_sc_gather = pl.kernel(
    _gather_body,
    mesh=plsc.VectorSubcoreMesh(core_axis_name="c", subcore_axis_name="s"),
    out_type=jax.ShapeDtypeStruct((BATCH, EMBED_DIM), jnp.float32),
    scratch_types=[
        pltpu.VMEM((B_PER_W,), jnp.int32),
        pltpu.VMEM((B_PER_W, EMBED_DIM), jnp.float32),
        pltpu.SemaphoreType.DMA,
    ],
    compiler_params=pltpu.CompilerParams(use_tc_tiling_on_sc=False),
)


TN = 4096


def _proj_body(w_ref, b_ref, emb_ref, out_ref):
    w_aug = jnp.concatenate([w_ref[...], b_ref[...]], axis=0)
    emb_aug = jnp.concatenate(
        [emb_ref[...], jnp.ones((BATCH, 1), jnp.float32)], axis=1)
    out_ref[...] = lax.dot_general(
        w_aug, emb_aug,
        dimension_numbers=(((0,), (1,)), ((), ())),
        preferred_element_type=jnp.float32,
    )


@jax.jit
def _run(idx, emb_table, W, b):
    embeds = _sc_gather(emb_table, idx)
    out_t = pl.pallas_call(
        _proj_body,
        grid=(pl.cdiv(VOCAB, TN),),
        in_specs=[
            pl.BlockSpec((EMBED_DIM, TN), lambda i: (0, i)),
            pl.BlockSpec((1, TN), lambda i: (0, i)),
            pl.BlockSpec((BATCH, EMBED_DIM), lambda i: (0, 0)),
        ],
        out_specs=pl.BlockSpec((TN, BATCH), lambda i: (i, 0)),
        out_shape=jax.ShapeDtypeStruct((VOCAB, BATCH), jnp.float32),
        compiler_params=pltpu.CompilerParams(
            dimension_semantics=("parallel",),
        ),
    )(W.T, b.reshape(1, VOCAB), embeds)
    return out_t.T


def kernel(inputs, emb_table, W, b):
    return _run(inputs.astype(jnp.int32), emb_table, W, b)
